# baseline (speedup 1.0000x reference)
import jax
import jax.numpy as jnp
from jax import lax
from jax.experimental import pallas as pl
from jax.experimental.pallas import tpu as pltpu
from jax.experimental.pallas import tpu_sc as plsc

B = 4096
D = 1024
O2 = 512
O3 = 512
L = 16
NC = 2
NS = 16
NW = NC * NS
RPW = B // NW
CB = 16
NCHUNK = RPW // CB
U2 = 16
U3 = 8


def _compute_chunk(xb, ob, ibuf):
    def j2(j, _):
        col = pl.ds(pl.multiple_of(j * L, L), L)
        ia = ibuf[pl.ds(j * L, L)]
        ib = ibuf[pl.ds(O2 + j * L, L)]
        for t in range(0, CB, U2):
            va = [plsc.load_gather(xb, [jnp.full((L,), t + u, jnp.int32), ia])
                  for u in range(U2)]
            vb = [plsc.load_gather(xb, [jnp.full((L,), t + u, jnp.int32), ib])
                  for u in range(U2)]
            for u in range(U2):
                ob[t + u, col] = va[u] * vb[u]
        return 0

    lax.fori_loop(0, O2 // L, j2, 0)

    def j3(j, _):
        col = pl.ds(pl.multiple_of(O2 + j * L, L), L)
        ia = ibuf[pl.ds(2 * O2 + j * L, L)]
        ib = ibuf[pl.ds(2 * O2 + O3 + j * L, L)]
        ic = ibuf[pl.ds(2 * O2 + 2 * O3 + j * L, L)]
        for t in range(0, CB, U3):
            va = [plsc.load_gather(xb, [jnp.full((L,), t + u, jnp.int32), ia])
                  for u in range(U3)]
            vb = [plsc.load_gather(xb, [jnp.full((L,), t + u, jnp.int32), ib])
                  for u in range(U3)]
            vc = [plsc.load_gather(xb, [jnp.full((L,), t + u, jnp.int32), ic])
                  for u in range(U3)]
            for u in range(U3):
                ob[t + u, col] = va[u] * vb[u] * vc[u]
        return 0

    lax.fori_loop(0, O3 // L, j3, 0)


def _body(x_hbm, iall_hbm, out_hbm, xbuf, obuf, ibuf,
          sin0, sin1, sout0, sout1, sidx):
    wid = lax.axis_index("s") * NC + lax.axis_index("c")
    idx_d = pltpu.async_copy(iall_hbm, ibuf, sidx)
    base0 = wid * RPW
    sin = [sin0, sin1]
    sout = [sout0, sout1]
    in_d = [None] * NCHUNK
    out_d = [None] * NCHUNK

    def start_in(ci):
        return pltpu.async_copy(
            x_hbm.at[pl.ds(base0 + ci * CB, CB)], xbuf.at[ci % 2], sin[ci % 2])

    in_d[0] = start_in(0)
    idx_d.wait()
    for ci in range(NCHUNK):
        in_d[ci].wait()
        if ci + 1 < NCHUNK:
            in_d[ci + 1] = start_in(ci + 1)
        if ci >= 2:
            out_d[ci - 2].wait()
        _compute_chunk(xbuf.at[ci % 2], obuf.at[ci % 2], ibuf)
        out_d[ci] = pltpu.async_copy(
            obuf.at[ci % 2], out_hbm.at[pl.ds(base0 + ci * CB, CB)],
            sout[ci % 2])
    out_d[NCHUNK - 2].wait()
    out_d[NCHUNK - 1].wait()


def kernel(x, idx0, idx1):
    iall = jnp.concatenate([idx0.T.reshape(-1), idx1.T.reshape(-1)])
    mesh = plsc.VectorSubcoreMesh(core_axis_name="c", subcore_axis_name="s")
    k = pl.kernel(
        _body,
        out_type=jax.ShapeDtypeStruct((B, D), jnp.float32),
        mesh=mesh,
        compiler_params=pltpu.CompilerParams(
            use_tc_tiling_on_sc=True, needs_layout_passes=False),
        scratch_types=[
            pltpu.VMEM((2, CB, D), jnp.float32),
            pltpu.VMEM((2, CB, D), jnp.float32),
            pltpu.VMEM((2 * O2 + 3 * O3,), jnp.int32),
            pltpu.SemaphoreType.DMA,
            pltpu.SemaphoreType.DMA,
            pltpu.SemaphoreType.DMA,
            pltpu.SemaphoreType.DMA,
            pltpu.SemaphoreType.DMA,
        ],
    )
    return k(x, iall)


# speedup vs baseline: 1.0319x; 1.0319x over previous
import jax
import jax.numpy as jnp
from jax import lax
from jax.experimental import pallas as pl
from jax.experimental.pallas import tpu as pltpu
from jax.experimental.pallas import tpu_sc as plsc

B = 4096
D = 1024
O2 = 512
O3 = 512
L = 16
NC = 2
NS = 16
NW = NC * NS
RPW = B // NW
CB = 16
NCHUNK = RPW // CB
U2 = 8
U3 = 8


def _compute_chunk(xb, ob, ibuf):
    @plsc.parallel_loop(0, O2 // L, 1, unroll=2)
    def j2(j):
        col = pl.ds(pl.multiple_of(j * L, L), L)
        ia = ibuf[pl.ds(j * L, L)]
        ib = ibuf[pl.ds(O2 + j * L, L)]
        for t in range(0, CB, U2):
            va = [plsc.load_gather(xb, [jnp.full((L,), t + u, jnp.int32), ia])
                  for u in range(U2)]
            vb = [plsc.load_gather(xb, [jnp.full((L,), t + u, jnp.int32), ib])
                  for u in range(U2)]
            for u in range(U2):
                ob[t + u, col] = va[u] * vb[u]

    @plsc.parallel_loop(0, O3 // L, 1, unroll=2)
    def j3(j):
        col = pl.ds(pl.multiple_of(O2 + j * L, L), L)
        ia = ibuf[pl.ds(2 * O2 + j * L, L)]
        ib = ibuf[pl.ds(2 * O2 + O3 + j * L, L)]
        ic = ibuf[pl.ds(2 * O2 + 2 * O3 + j * L, L)]
        for t in range(0, CB, U3):
            va = [plsc.load_gather(xb, [jnp.full((L,), t + u, jnp.int32), ia])
                  for u in range(U3)]
            vb = [plsc.load_gather(xb, [jnp.full((L,), t + u, jnp.int32), ib])
                  for u in range(U3)]
            vc = [plsc.load_gather(xb, [jnp.full((L,), t + u, jnp.int32), ic])
                  for u in range(U3)]
            for u in range(U3):
                ob[t + u, col] = va[u] * vb[u] * vc[u]


def _body(x_hbm, iall_hbm, out_hbm, xbuf, obuf, ibuf,
          sin0, sin1, sout0, sout1, sidx):
    wid = lax.axis_index("s") * NC + lax.axis_index("c")
    idx_d = pltpu.async_copy(iall_hbm, ibuf, sidx)
    base0 = wid * RPW
    sin = [sin0, sin1]
    sout = [sout0, sout1]
    in_d = [None] * NCHUNK
    out_d = [None] * NCHUNK

    def start_in(ci):
        return pltpu.async_copy(
            x_hbm.at[pl.ds(base0 + ci * CB, CB)], xbuf.at[ci % 2], sin[ci % 2])

    in_d[0] = start_in(0)
    idx_d.wait()
    for ci in range(NCHUNK):
        in_d[ci].wait()
        if ci + 1 < NCHUNK:
            in_d[ci + 1] = start_in(ci + 1)
        if ci >= 2:
            out_d[ci - 2].wait()
        _compute_chunk(xbuf.at[ci % 2], obuf.at[ci % 2], ibuf)
        out_d[ci] = pltpu.async_copy(
            obuf.at[ci % 2], out_hbm.at[pl.ds(base0 + ci * CB, CB)],
            sout[ci % 2])
    out_d[NCHUNK - 2].wait()
    out_d[NCHUNK - 1].wait()


def kernel(x, idx0, idx1):
    iall = jnp.concatenate([idx0.T.reshape(-1), idx1.T.reshape(-1)])
    mesh = plsc.VectorSubcoreMesh(core_axis_name="c", subcore_axis_name="s")
    k = pl.kernel(
        _body,
        out_type=jax.ShapeDtypeStruct((B, D), jnp.float32),
        mesh=mesh,
        compiler_params=pltpu.CompilerParams(
            use_tc_tiling_on_sc=True, needs_layout_passes=False),
        scratch_types=[
            pltpu.VMEM((2, CB, D), jnp.float32),
            pltpu.VMEM((2, CB, D), jnp.float32),
            pltpu.VMEM((2 * O2 + 3 * O3,), jnp.int32),
            pltpu.SemaphoreType.DMA,
            pltpu.SemaphoreType.DMA,
            pltpu.SemaphoreType.DMA,
            pltpu.SemaphoreType.DMA,
            pltpu.SemaphoreType.DMA,
        ],
    )
    return k(x, iall)
